# baseline (speedup 1.0000x reference)
import functools
import jax
import jax.numpy as jnp
from jax import lax
from jax.experimental import pallas as pl
from jax.experimental.pallas import tpu as pltpu
from jax.experimental.pallas import tpu_sc as plsc

B, N, D_IN, D_OUT = 16, 4096, 64, 64
SEG = 4
SROWS = D_IN // SEG
NSLOT = 16
GRP = 8
NGRP = 32 // GRP


def _sc_max_body(feat_hbm, nu_hbm, out_hbm, buf, nuv, res):
    cid = lax.axis_index("c")
    sid = lax.axis_index("s")
    wid = sid * 2 + cid
    b = wid // 2
    h = wid % 2
    pltpu.sync_copy(nu_hbm, nuv)
    iota16 = lax.broadcasted_iota(jnp.int32, (16,), 0)
    nu = nuv[pl.ds(b, 1)][0]
    nfull = nu // 16
    rem = nu - nfull * 16
    tail_off = jnp.minimum(nfull * 16, N - 16)
    for g in range(NGRP):
        d0 = h * 32 + g * GRP
        pltpu.sync_copy(feat_hbm.at[b, pl.ds(d0, GRP), :], buf)
        for r in range(GRP):
            def step(j, acc):
                return jnp.maximum(acc, buf[r, pl.ds(j * 16, 16)])
            acc = lax.fori_loop(0, nfull, step,
                                jnp.full((16,), -jnp.inf, dtype=jnp.float32))
            tail = jnp.where(iota16 < rem, buf[r, pl.ds(tail_off, 16)],
                             -jnp.inf)
            res[g * GRP + r] = jnp.maximum(acc, tail)
    pltpu.sync_copy(res, out_hbm.at[b, h])


def _sc_max(ft, num_unit):
    mesh = plsc.VectorSubcoreMesh(core_axis_name="c", subcore_axis_name="s")
    kern = functools.partial(
        pl.kernel,
        mesh=mesh,
        out_type=jax.ShapeDtypeStruct((B, 2, 32, 16), jnp.float32),
        scratch_types=[
            pltpu.VMEM((GRP, N), jnp.float32),
            pltpu.VMEM((16,), jnp.int32),
            pltpu.VMEM((32, 16), jnp.float32),
        ],
    )(_sc_max_body)
    return kern(ft, num_unit)


def _tc_body(g_ref, feat_hbm, fmax_ref, v_ref, b_ref, out_hbm,
             inbuf, outbuf, fsem, osem):
    def in_copy(b, slot, s):
        return pltpu.make_async_copy(
            feat_hbm.at[b, pl.ds(s * SROWS, SROWS), :],
            inbuf.at[slot, pl.ds(s * SROWS, SROWS), :],
            fsem.at[slot, s])

    def out_copy(b, slot, s):
        return pltpu.make_async_copy(
            outbuf.at[slot, pl.ds(s * SROWS, SROWS), :],
            out_hbm.at[b, pl.ds(s * SROWS, SROWS), :],
            osem.at[slot, s])

    v = v_ref[...]
    norm = jnp.sqrt(jnp.sum(v * v))
    w = v * (g_ref[0] / norm)
    bias = b_ref[...]

    for b in range(min(NSLOT - 1, B)):
        for s in range(SEG):
            in_copy(b, b % NSLOT, s).start()

    for b in range(B):
        slot = b % NSLOT
        nxt = b + NSLOT - 1
        if nxt < B:
            for s in range(SEG):
                in_copy(nxt, nxt % NSLOT, s).start()
        for s in range(SEG):
            in_copy(b, slot, s).wait()
        if b >= NSLOT:
            for s in range(SEG):
                out_copy(b - NSLOT, slot, s).wait()
        x = inbuf[slot]
        fm = jnp.reshape(fmax_ref[b], (D_IN, 16))
        fmax = jnp.max(fm, axis=1, keepdims=True)
        adj = bias - lax.dot_general(w, fmax, (((1,), (0,)), ((), ())),
                                     preferred_element_type=jnp.float32)
        out = lax.dot_general(w, x, (((1,), (0,)), ((), ())),
                              preferred_element_type=jnp.float32)
        outbuf[slot] = jnp.maximum(out + adj, 0.0)
        for s in range(SEG):
            out_copy(b, slot, s).start()

    for b in range(max(B - NSLOT, 0), B):
        for s in range(SEG):
            out_copy(b, b % NSLOT, s).wait()


def kernel(feat, num_unit, v, g, b):
    ft = jnp.transpose(feat, (0, 2, 1))
    partial = _sc_max(ft, num_unit)
    g2 = jnp.reshape(g, (1,))
    b2 = jnp.reshape(b, (D_OUT, 1))
    out_t = pl.pallas_call(
        _tc_body,
        grid=(),
        in_specs=[
            pl.BlockSpec(memory_space=pltpu.SMEM),
            pl.BlockSpec(memory_space=pl.ANY),
            pl.BlockSpec(memory_space=pltpu.VMEM),
            pl.BlockSpec(memory_space=pltpu.VMEM),
            pl.BlockSpec(memory_space=pltpu.VMEM),
        ],
        out_specs=pl.BlockSpec(memory_space=pl.ANY),
        out_shape=jax.ShapeDtypeStruct((B, D_OUT, N), jnp.float32),
        scratch_shapes=[
            pltpu.VMEM((NSLOT, D_IN, N), jnp.float32),
            pltpu.VMEM((NSLOT, D_OUT, N), jnp.float32),
            pltpu.SemaphoreType.DMA((NSLOT, SEG)),
            pltpu.SemaphoreType.DMA((NSLOT, SEG)),
        ],
    )(g2, ft, partial, v, b2)
    return jnp.transpose(out_t, (0, 2, 1))


# speedup vs baseline: 4.5896x; 4.5896x over previous
import jax
import jax.numpy as jnp
from jax import lax
from jax.experimental import pallas as pl
from jax.experimental.pallas import tpu as pltpu

B, N, D_IN, D_OUT = 16, 4096, 64, 64
SEG = 8
SROWS = D_IN // SEG
NSLOT = 16


def _body(nu_ref, g_ref, feat_hbm, v_ref, b_ref, out_hbm,
          inbuf, outbuf, fsem, osem):
    def in_copy(b, slot, s):
        return pltpu.make_async_copy(
            feat_hbm.at[b, pl.ds(s * SROWS, SROWS), :],
            inbuf.at[slot, pl.ds(s * SROWS, SROWS), :],
            fsem.at[slot, s])

    def out_copy(b, slot, s):
        return pltpu.make_async_copy(
            outbuf.at[slot, pl.ds(s * SROWS, SROWS), :],
            out_hbm.at[b, pl.ds(s * SROWS, SROWS), :],
            osem.at[slot, s])

    v = v_ref[...]
    norm = jnp.sqrt(jnp.sum(v * v))
    w = v * (g_ref[0] / norm)
    bias = b_ref[...]
    lane = lax.broadcasted_iota(jnp.int32, (1, N), 1)

    for b in range(min(NSLOT - 1, B)):
        for s in range(SEG):
            in_copy(b, b % NSLOT, s).start()

    for b in range(B):
        slot = b % NSLOT
        nxt = b + NSLOT - 1
        if nxt < B:
            for s in range(SEG):
                in_copy(nxt, nxt % NSLOT, s).start()
        for s in range(SEG):
            in_copy(b, slot, s).wait()
        if b >= NSLOT:
            for s in range(SEG):
                out_copy(b - NSLOT, slot, s).wait()
        x = inbuf[slot]
        pen = jnp.where(lane < nu_ref[b], 0.0, -jnp.inf)
        fmax = jnp.max(x + pen, axis=1, keepdims=True)
        adj = bias - lax.dot_general(w, fmax, (((1,), (0,)), ((), ())),
                                     preferred_element_type=jnp.float32)
        out = lax.dot_general(w, x, (((1,), (0,)), ((), ())),
                              preferred_element_type=jnp.float32)
        outbuf[slot] = jnp.maximum(out + adj, 0.0)
        for s in range(SEG):
            out_copy(b, slot, s).start()

    for b in range(max(B - NSLOT, 0), B):
        for s in range(SEG):
            out_copy(b, b % NSLOT, s).wait()


def kernel(feat, num_unit, v, g, b):
    ft = jnp.transpose(feat, (0, 2, 1))
    g2 = jnp.reshape(g, (1,))
    b2 = jnp.reshape(b, (D_OUT, 1))
    out_t = pl.pallas_call(
        _body,
        grid=(),
        in_specs=[
            pl.BlockSpec(memory_space=pltpu.SMEM),
            pl.BlockSpec(memory_space=pltpu.SMEM),
            pl.BlockSpec(memory_space=pl.ANY),
            pl.BlockSpec(memory_space=pltpu.VMEM),
            pl.BlockSpec(memory_space=pltpu.VMEM),
        ],
        out_specs=pl.BlockSpec(memory_space=pl.ANY),
        out_shape=jax.ShapeDtypeStruct((B, D_OUT, N), jnp.float32),
        scratch_shapes=[
            pltpu.VMEM((NSLOT, D_IN, N), jnp.float32),
            pltpu.VMEM((NSLOT, D_OUT, N), jnp.float32),
            pltpu.SemaphoreType.DMA((NSLOT, SEG)),
            pltpu.SemaphoreType.DMA((NSLOT, SEG)),
        ],
    )(num_unit, g2, ft, v, b2)
    return jnp.transpose(out_t, (0, 2, 1))


# speedup vs baseline: 4.9236x; 1.0728x over previous
import jax
import jax.numpy as jnp
from jax import lax
from jax.experimental import pallas as pl
from jax.experimental.pallas import tpu as pltpu

B, N, D_IN, D_OUT = 16, 4096, 64, 64
SEG = 4
SROWS = D_IN // SEG
NSLOT = 16


def _body(nu_ref, g_ref, feat_hbm, v_ref, b_ref, out_hbm,
          inbuf, outbuf, fsem, osem):
    def in_copy(b, slot, s):
        return pltpu.make_async_copy(
            feat_hbm.at[b, pl.ds(s * SROWS, SROWS), :],
            inbuf.at[slot, pl.ds(s * SROWS, SROWS), :],
            fsem.at[slot, s])

    def out_copy(b, slot, s):
        return pltpu.make_async_copy(
            outbuf.at[slot, pl.ds(s * SROWS, SROWS), :],
            out_hbm.at[b, pl.ds(s * SROWS, SROWS), :],
            osem.at[slot, s])

    v = v_ref[...]
    norm = jnp.sqrt(jnp.sum(v * v))
    w = v * (g_ref[0] / norm)
    bias = b_ref[...]
    lane = lax.broadcasted_iota(jnp.int32, (1, N), 1)

    for b in range(min(NSLOT - 1, B)):
        for s in range(SEG):
            in_copy(b, b % NSLOT, s).start()

    for b in range(B):
        slot = b % NSLOT
        nxt = b + NSLOT - 1
        if nxt < B:
            for s in range(SEG):
                in_copy(nxt, nxt % NSLOT, s).start()
        for s in range(SEG):
            in_copy(b, slot, s).wait()
        if b >= NSLOT:
            for s in range(SEG):
                out_copy(b - NSLOT, slot, s).wait()
        x = inbuf[slot]
        pen = jnp.where(lane < nu_ref[b], 0.0, -jnp.inf)
        fmax = jnp.max(x + pen, axis=1, keepdims=True)
        adj = bias - lax.dot_general(w, fmax, (((1,), (0,)), ((), ())),
                                     preferred_element_type=jnp.float32)
        out = lax.dot_general(w, x, (((1,), (0,)), ((), ())),
                              preferred_element_type=jnp.float32)
        outbuf[slot] = jnp.maximum(out + adj, 0.0)
        for s in range(SEG):
            out_copy(b, slot, s).start()

    for b in range(max(B - NSLOT, 0), B):
        for s in range(SEG):
            out_copy(b, b % NSLOT, s).wait()


def kernel(feat, num_unit, v, g, b):
    ft = jnp.transpose(feat, (0, 2, 1))
    g2 = jnp.reshape(g, (1,))
    b2 = jnp.reshape(b, (D_OUT, 1))
    out_t = pl.pallas_call(
        _body,
        grid=(),
        in_specs=[
            pl.BlockSpec(memory_space=pltpu.SMEM),
            pl.BlockSpec(memory_space=pltpu.SMEM),
            pl.BlockSpec(memory_space=pl.ANY),
            pl.BlockSpec(memory_space=pltpu.VMEM),
            pl.BlockSpec(memory_space=pltpu.VMEM),
        ],
        out_specs=pl.BlockSpec(memory_space=pl.ANY),
        out_shape=jax.ShapeDtypeStruct((B, D_OUT, N), jnp.float32),
        scratch_shapes=[
            pltpu.VMEM((NSLOT, D_IN, N), jnp.float32),
            pltpu.VMEM((NSLOT, D_OUT, N), jnp.float32),
            pltpu.SemaphoreType.DMA((NSLOT, SEG)),
            pltpu.SemaphoreType.DMA((NSLOT, SEG)),
        ],
    )(num_unit, g2, ft, v, b2)
    return jnp.transpose(out_t, (0, 2, 1))


# speedup vs baseline: 4.9561x; 1.0066x over previous
import jax
import jax.numpy as jnp
from jax import lax
from jax.experimental import pallas as pl
from jax.experimental.pallas import tpu as pltpu

B, N, D_IN, D_OUT = 16, 4096, 64, 64
SEG = 4
SROWS = D_IN // SEG
NSLOT = 16


def _body(nu_ref, g_ref, feat_hbm, v_ref, b_ref, out_hbm,
          inbuf, outbuf, fsem, osem):
    def in_copy(b, slot, s):
        return pltpu.make_async_copy(
            feat_hbm.at[b, pl.ds(s * SROWS, SROWS), :],
            inbuf.at[slot, pl.ds(s * SROWS, SROWS), :],
            fsem.at[slot, s])

    def out_copy(b, slot, s):
        return pltpu.make_async_copy(
            outbuf.at[slot, pl.ds(s * SROWS, SROWS), :],
            out_hbm.at[b, pl.ds(s * SROWS, SROWS), :],
            osem.at[slot, s])

    v = v_ref[...]
    norm = jnp.sqrt(jnp.sum(v * v))
    w = v * (g_ref[0] / norm)
    bias = b_ref[...]
    lane = lax.broadcasted_iota(jnp.int32, (1, N), 1)

    for b in range(min(NSLOT - 1, B)):
        for s in range(SEG):
            in_copy(b, b % NSLOT, s).start()

    for bb in range(0, B, 2):
        pair = (bb, bb + 1)
        for b in pair:
            slot = b % NSLOT
            nxt = b + NSLOT - 1
            if nxt < B:
                for s in range(SEG):
                    in_copy(nxt, nxt % NSLOT, s).start()
            for s in range(SEG):
                in_copy(b, slot, s).wait()
            if b >= NSLOT:
                for s in range(SEG):
                    out_copy(b - NSLOT, slot, s).wait()
        for b in pair:
            slot = b % NSLOT
            x = inbuf[slot]
            pen = jnp.where(lane < nu_ref[b], 0.0, -jnp.inf)
            fmax = jnp.max(x + pen, axis=1, keepdims=True)
            adj = bias - lax.dot_general(w, fmax, (((1,), (0,)), ((), ())),
                                         preferred_element_type=jnp.float32)
            out = lax.dot_general(w, x, (((1,), (0,)), ((), ())),
                                  preferred_element_type=jnp.float32)
            outbuf[slot] = jnp.maximum(out + adj, 0.0)
            for s in range(SEG):
                out_copy(b, slot, s).start()

    for b in range(max(B - NSLOT, 0), B):
        for s in range(SEG):
            out_copy(b, b % NSLOT, s).wait()


def kernel(feat, num_unit, v, g, b):
    ft = jnp.transpose(feat, (0, 2, 1))
    g2 = jnp.reshape(g, (1,))
    b2 = jnp.reshape(b, (D_OUT, 1))
    out_t = pl.pallas_call(
        _body,
        grid=(),
        in_specs=[
            pl.BlockSpec(memory_space=pltpu.SMEM),
            pl.BlockSpec(memory_space=pltpu.SMEM),
            pl.BlockSpec(memory_space=pl.ANY),
            pl.BlockSpec(memory_space=pltpu.VMEM),
            pl.BlockSpec(memory_space=pltpu.VMEM),
        ],
        out_specs=pl.BlockSpec(memory_space=pl.ANY),
        out_shape=jax.ShapeDtypeStruct((B, D_OUT, N), jnp.float32),
        scratch_shapes=[
            pltpu.VMEM((NSLOT, D_IN, N), jnp.float32),
            pltpu.VMEM((NSLOT, D_OUT, N), jnp.float32),
            pltpu.SemaphoreType.DMA((NSLOT, SEG)),
            pltpu.SemaphoreType.DMA((NSLOT, SEG)),
        ],
    )(num_unit, g2, ft, v, b2)
    return jnp.transpose(out_t, (0, 2, 1))
